# baseline (speedup 1.0000x reference)
import functools

import jax
import jax.numpy as jnp
from jax import lax
from jax.experimental import pallas as pl
from jax.experimental.pallas import tpu as pltpu
from jax.experimental.pallas import tpu_sc as plsc

N_NODES = 10000
N_EDGES = 320000
DIM = 128
HALF = DIM // 2

NC = 2
NS = 16
CHUNK = 128
CHUNKS_PER_TILE = 80
EDGES_PER_TILE = CHUNK * CHUNKS_PER_TILE
E_PAD = NC * NS * EDGES_PER_TILE
ACC_ROWS = 10112
ROWS_PER_TILE = ACC_ROWS // NS
X_LAST_TILE = N_NODES - 15 * ROWS_PER_TILE

NSTREAM = 2
CPS = CHUNKS_PER_TILE // NSTREAM


def _sc_aggregate(x, src_t, dst_t, zeros_init):
  mesh = plsc.VectorSubcoreMesh(
      core_axis_name="c", subcore_axis_name="s", num_cores=NC,
      num_subcores=NS)

  @functools.partial(
      pl.kernel,
      out_type=jax.ShapeDtypeStruct((NC, 2, ACC_ROWS, HALF), jnp.float32),
      mesh=mesh,
      scratch_types=[
          pltpu.VMEM_SHARED((ACC_ROWS, HALF), jnp.float32),
          pltpu.VMEM_SHARED((ACC_ROWS, HALF), jnp.float32),
          pltpu.VMEM((CHUNKS_PER_TILE, CHUNK), jnp.int32),
          pltpu.VMEM((CHUNKS_PER_TILE, CHUNK), jnp.int32),
          pltpu.VMEM((NSTREAM, CHUNK, HALF), jnp.float32),
          [pltpu.SemaphoreType.DMA] * NSTREAM,
          [pltpu.SemaphoreType.DMA] * NSTREAM,
      ],
      compiler_params=pltpu.CompilerParams(use_tc_tiling_on_sc=False),
  )
  def sc_kernel(x_hbm, src_hbm, dst_hbm, zer_hbm, out_hbm,
                x_sp, acc, src_v, dst_v, rows, gsems, ssems):
    c = lax.axis_index("c")
    s = lax.axis_index("s")
    rslice = pl.ds(s * ROWS_PER_TILE, ROWS_PER_TILE)

    pltpu.sync_copy(src_hbm.at[c, s], src_v)
    pltpu.sync_copy(dst_hbm.at[c, s], dst_v)

    def fire_gather(t, i):
      pltpu.async_copy(x_sp.at[src_v.at[t * CPS + i]], rows.at[t],
                       gsems[t])

    def wait_gather(t, i):
      pltpu.make_async_copy(x_sp.at[src_v.at[t * CPS + i]],
                            rows.at[t], gsems[t]).wait()

    def fire_scatter(t, i):
      pltpu.async_copy(rows.at[t], acc.at[dst_v.at[t * CPS + i]],
                       ssems[t], add=True)

    def wait_scatter(t, i):
      pltpu.make_async_copy(rows.at[t], acc.at[dst_v.at[t * CPS + i]],
                            ssems[t]).wait()

    for p in range(2):
      cslice = pl.ds(p * HALF, HALF)
      @pl.when(s < NS - 1)
      def _():
        pltpu.sync_copy(
            x_hbm.at[pl.ds(s * ROWS_PER_TILE, ROWS_PER_TILE), cslice],
            x_sp.at[rslice])
      @pl.when(s == NS - 1)
      def _():
        pltpu.sync_copy(
            x_hbm.at[pl.ds(15 * ROWS_PER_TILE, X_LAST_TILE), cslice],
            x_sp.at[pl.ds(15 * ROWS_PER_TILE, X_LAST_TILE)])
      pltpu.sync_copy(zer_hbm.at[rslice], acc.at[rslice])
      plsc.subcore_barrier()

      for t in range(NSTREAM):
        fire_gather(t, 0)

      @pl.loop(0, CPS)
      def _(i):
        for t in range(NSTREAM):
          wait_gather(t, i)
          fire_scatter(t, i)
        for t in range(NSTREAM):
          wait_scatter(t, i)
          @pl.when(i < CPS - 1)
          def _():
            fire_gather(t, i + 1)

      plsc.subcore_barrier()
      pltpu.sync_copy(acc.at[rslice], out_hbm.at[c, p, rslice])
      plsc.subcore_barrier()

  return sc_kernel(x, src_t, dst_t, zeros_init)


def _tc_body(a_hbm, w1_ref, b1_ref, w2_ref, b2_ref, o_ref, a_ref, sem):
  pltpu.async_copy(a_hbm, a_ref, sem).wait()
  a_lo = a_ref[0, 0, :N_NODES] + a_ref[1, 0, :N_NODES]
  a_hi = a_ref[0, 1, :N_NODES] + a_ref[1, 1, :N_NODES]
  w1_lo = w1_ref[:, :HALF]
  w1_hi = w1_ref[:, HALF:]
  h = (lax.dot_general(a_lo, w1_lo, (((1,), (1,)), ((), ())),
                       preferred_element_type=jnp.float32)
       + lax.dot_general(a_hi, w1_hi, (((1,), (1,)), ((), ())),
                         preferred_element_type=jnp.float32))
  h = jnp.maximum(h + b1_ref[...], 0.0)
  o = lax.dot_general(w2_ref[...], h, (((1,), (1,)), ((), ())),
                      preferred_element_type=jnp.float32)
  o_ref[...] = o + b2_ref[...]


def _tc_linear(agg4, W1, b1, W2, b2):
  return pl.pallas_call(
      _tc_body,
      out_shape=jax.ShapeDtypeStruct((DIM, N_NODES), jnp.float32),
      in_specs=[
          pl.BlockSpec(memory_space=pl.ANY),
          pl.BlockSpec((DIM, DIM), lambda: (0, 0)),
          pl.BlockSpec((1, DIM), lambda: (0, 0)),
          pl.BlockSpec((DIM, DIM), lambda: (0, 0)),
          pl.BlockSpec((DIM, 1), lambda: (0, 0)),
      ],
      scratch_shapes=[
          pltpu.VMEM((NC, 2, ACC_ROWS, HALF), jnp.float32),
          pltpu.SemaphoreType.DMA,
      ],
  )(agg4, W1, b1.reshape(1, DIM), W2, b2.reshape(DIM, 1))


def kernel(x, edge_index, W1, b1, W2, b2):
  src = edge_index[0]
  dst = edge_index[1]
  pad = E_PAD - N_EDGES
  src_t = jnp.concatenate(
      [src, jnp.zeros((pad,), jnp.int32)]).reshape(
          NC, NS, CHUNKS_PER_TILE, CHUNK)
  dst_t = jnp.concatenate(
      [dst, jnp.full((pad,), N_NODES, jnp.int32)]).reshape(
          NC, NS, CHUNKS_PER_TILE, CHUNK)
  zeros_init = jnp.zeros((ACC_ROWS, HALF), jnp.float32)
  agg4 = _sc_aggregate(x, src_t, dst_t, zeros_init)
  return _tc_linear(agg4, W1, b1, W2, b2)


# speedup vs baseline: 1.0916x; 1.0916x over previous
import functools

import jax
import jax.numpy as jnp
from jax import lax
from jax.experimental import pallas as pl
from jax.experimental.pallas import tpu as pltpu
from jax.experimental.pallas import tpu_sc as plsc

N_NODES = 10000
N_EDGES = 320000
DIM = 128
HALF = DIM // 2

NC = 2
NS = 16
CHUNK = 128
CHUNKS_PER_TILE = 80
EDGES_PER_TILE = CHUNK * CHUNKS_PER_TILE
E_PAD = NC * NS * EDGES_PER_TILE
ACC_ROWS = 10112
ROWS_PER_TILE = ACC_ROWS // NS
X_LAST_TILE = N_NODES - 15 * ROWS_PER_TILE

NSTREAM = 2
CPS = CHUNKS_PER_TILE // NSTREAM
TAIL_CHUNKS = N_EDGES // CHUNK - (NC * NS - 1) * CHUNKS_PER_TILE


def _sc_aggregate(x, e3, zeros_init):
  mesh = plsc.VectorSubcoreMesh(
      core_axis_name="c", subcore_axis_name="s", num_cores=NC,
      num_subcores=NS)

  @functools.partial(
      pl.kernel,
      out_type=jax.ShapeDtypeStruct((NC, 2, ACC_ROWS, HALF), jnp.float32),
      mesh=mesh,
      scratch_types=[
          pltpu.VMEM_SHARED((ACC_ROWS, HALF), jnp.float32),
          pltpu.VMEM_SHARED((ACC_ROWS, HALF), jnp.float32),
          pltpu.VMEM((CHUNKS_PER_TILE, CHUNK), jnp.int32),
          pltpu.VMEM((CHUNKS_PER_TILE, CHUNK), jnp.int32),
          pltpu.VMEM((NSTREAM, CHUNK, HALF), jnp.float32),
          [pltpu.SemaphoreType.DMA] * NSTREAM,
          [pltpu.SemaphoreType.DMA] * NSTREAM,
      ],
      compiler_params=pltpu.CompilerParams(use_tc_tiling_on_sc=False),
  )
  def sc_kernel(x_hbm, e3_hbm, zer_hbm, out_hbm,
                x_sp, acc, src_v, dst_v, rows, gsems, ssems):
    c = lax.axis_index("c")
    s = lax.axis_index("s")
    rslice = pl.ds(s * ROWS_PER_TILE, ROWS_PER_TILE)
    w = c * NS + s
    cps = lax.select(w == NC * NS - 1, TAIL_CHUNKS // NSTREAM, CPS)

    @pl.when(w < NC * NS - 1)
    def _():
      pltpu.sync_copy(e3_hbm.at[0, pl.ds(w * CHUNKS_PER_TILE,
                                         CHUNKS_PER_TILE)], src_v)
      pltpu.sync_copy(e3_hbm.at[1, pl.ds(w * CHUNKS_PER_TILE,
                                         CHUNKS_PER_TILE)], dst_v)
    @pl.when(w == NC * NS - 1)
    def _():
      base = (NC * NS - 1) * CHUNKS_PER_TILE
      pltpu.sync_copy(e3_hbm.at[0, pl.ds(base, TAIL_CHUNKS)],
                      src_v.at[pl.ds(0, TAIL_CHUNKS)])
      pltpu.sync_copy(e3_hbm.at[1, pl.ds(base, TAIL_CHUNKS)],
                      dst_v.at[pl.ds(0, TAIL_CHUNKS)])

    def fire_gather(t, i):
      pltpu.async_copy(x_sp.at[src_v.at[t * cps + i]], rows.at[t],
                       gsems[t])

    def wait_gather(t, i):
      pltpu.make_async_copy(x_sp.at[src_v.at[t * cps + i]],
                            rows.at[t], gsems[t]).wait()

    def fire_scatter(t, i):
      pltpu.async_copy(rows.at[t], acc.at[dst_v.at[t * cps + i]],
                       ssems[t], add=True)

    def wait_scatter(t, i):
      pltpu.make_async_copy(rows.at[t], acc.at[dst_v.at[t * cps + i]],
                            ssems[t]).wait()

    for p in range(2):
      cslice = pl.ds(p * HALF, HALF)
      @pl.when(s < NS - 1)
      def _():
        pltpu.sync_copy(
            x_hbm.at[pl.ds(s * ROWS_PER_TILE, ROWS_PER_TILE), cslice],
            x_sp.at[rslice])
      @pl.when(s == NS - 1)
      def _():
        pltpu.sync_copy(
            x_hbm.at[pl.ds(15 * ROWS_PER_TILE, X_LAST_TILE), cslice],
            x_sp.at[pl.ds(15 * ROWS_PER_TILE, X_LAST_TILE)])
      pltpu.sync_copy(zer_hbm.at[rslice], acc.at[rslice])
      plsc.subcore_barrier()

      for t in range(NSTREAM):
        fire_gather(t, 0)

      @pl.loop(0, cps)
      def _(i):
        for t in range(NSTREAM):
          wait_gather(t, i)
          fire_scatter(t, i)
        for t in range(NSTREAM):
          wait_scatter(t, i)
          @pl.when(i < cps - 1)
          def _():
            fire_gather(t, i + 1)

      plsc.subcore_barrier()
      pltpu.sync_copy(acc.at[rslice], out_hbm.at[c, p, rslice])
      plsc.subcore_barrier()

  return sc_kernel(x, e3, zeros_init)


def _tc_body(a_hbm, w1_ref, b1_ref, w2_ref, b2_ref, o_ref, a_ref, sem):
  pltpu.async_copy(a_hbm, a_ref, sem).wait()
  a_lo = a_ref[0, 0, :N_NODES] + a_ref[1, 0, :N_NODES]
  a_hi = a_ref[0, 1, :N_NODES] + a_ref[1, 1, :N_NODES]
  w1_lo = w1_ref[:, :HALF]
  w1_hi = w1_ref[:, HALF:]
  h = (lax.dot_general(a_lo, w1_lo, (((1,), (1,)), ((), ())),
                       preferred_element_type=jnp.float32)
       + lax.dot_general(a_hi, w1_hi, (((1,), (1,)), ((), ())),
                         preferred_element_type=jnp.float32))
  h = jnp.maximum(h + b1_ref[...], 0.0)
  o = lax.dot_general(w2_ref[...], h, (((1,), (1,)), ((), ())),
                      preferred_element_type=jnp.float32)
  o_ref[...] = o + b2_ref[...]


def _tc_linear(agg4, W1, b1, W2, b2):
  return pl.pallas_call(
      _tc_body,
      out_shape=jax.ShapeDtypeStruct((DIM, N_NODES), jnp.float32),
      in_specs=[
          pl.BlockSpec(memory_space=pl.ANY),
          pl.BlockSpec((DIM, DIM), lambda: (0, 0)),
          pl.BlockSpec((1, DIM), lambda: (0, 0)),
          pl.BlockSpec((DIM, DIM), lambda: (0, 0)),
          pl.BlockSpec((DIM, 1), lambda: (0, 0)),
      ],
      scratch_shapes=[
          pltpu.VMEM((NC, 2, ACC_ROWS, HALF), jnp.float32),
          pltpu.SemaphoreType.DMA,
      ],
  )(agg4, W1, b1.reshape(1, DIM), W2, b2.reshape(DIM, 1))


def kernel(x, edge_index, W1, b1, W2, b2):
  e3 = edge_index.reshape(2, N_EDGES // CHUNK, CHUNK)
  zeros_init = jnp.zeros((ACC_ROWS, HALF), jnp.float32)
  agg4 = _sc_aggregate(x, e3, zeros_init)
  return _tc_linear(agg4, W1, b1, W2, b2)


# speedup vs baseline: 1.0922x; 1.0006x over previous
import functools

import jax
import jax.numpy as jnp
from jax import lax
from jax.experimental import pallas as pl
from jax.experimental.pallas import tpu as pltpu
from jax.experimental.pallas import tpu_sc as plsc

N_NODES = 10000
N_EDGES = 320000
DIM = 128
HALF = DIM // 2

NC = 2
NS = 16
CHUNK = 128
CHUNKS_PER_TILE = 80
ACC_ROWS = 10112
ROWS_PER_TILE = ACC_ROWS // NS
X_LAST_TILE = N_NODES - 15 * ROWS_PER_TILE

NSTREAM = 2
CPS = CHUNKS_PER_TILE // NSTREAM
TAIL_CHUNKS = N_EDGES // CHUNK - (NC * NS - 1) * CHUNKS_PER_TILE


def _sc_aggregate(x, e3, zeros_init):
  mesh = plsc.VectorSubcoreMesh(
      core_axis_name="c", subcore_axis_name="s", num_cores=NC,
      num_subcores=NS)

  @functools.partial(
      pl.kernel,
      out_type=jax.ShapeDtypeStruct((NC, 2, ACC_ROWS, HALF), jnp.float32),
      mesh=mesh,
      scratch_types=[
          pltpu.VMEM_SHARED((ACC_ROWS, HALF), jnp.float32),
          pltpu.VMEM_SHARED((ACC_ROWS, HALF), jnp.float32),
          pltpu.VMEM((CHUNKS_PER_TILE, CHUNK), jnp.int32),
          pltpu.VMEM((CHUNKS_PER_TILE, CHUNK), jnp.int32),
          pltpu.VMEM((NSTREAM, CHUNK, HALF), jnp.float32),
          [pltpu.SemaphoreType.DMA] * NSTREAM,
          [pltpu.SemaphoreType.DMA] * NSTREAM,
      ],
      compiler_params=pltpu.CompilerParams(use_tc_tiling_on_sc=False),
  )
  def sc_kernel(x_hbm, e3_hbm, zer_hbm, out_hbm,
                x_sp, acc, src_v, dst_v, rows, gsems, ssems):
    c = lax.axis_index("c")
    s = lax.axis_index("s")
    rslice = pl.ds(s * ROWS_PER_TILE, ROWS_PER_TILE)
    w = c * NS + s
    cps = lax.select(w == NC * NS - 1, TAIL_CHUNKS // NSTREAM, CPS)

    @pl.when(w < NC * NS - 1)
    def _():
      pltpu.sync_copy(e3_hbm.at[0, pl.ds(w * CHUNKS_PER_TILE,
                                         CHUNKS_PER_TILE)], src_v)
      pltpu.sync_copy(e3_hbm.at[1, pl.ds(w * CHUNKS_PER_TILE,
                                         CHUNKS_PER_TILE)], dst_v)
    @pl.when(w == NC * NS - 1)
    def _():
      base = (NC * NS - 1) * CHUNKS_PER_TILE
      pltpu.sync_copy(e3_hbm.at[0, pl.ds(base, TAIL_CHUNKS)],
                      src_v.at[pl.ds(0, TAIL_CHUNKS)])
      pltpu.sync_copy(e3_hbm.at[1, pl.ds(base, TAIL_CHUNKS)],
                      dst_v.at[pl.ds(0, TAIL_CHUNKS)])

    def fire_gather(t, i):
      pltpu.async_copy(x_sp.at[src_v.at[t * cps + i]], rows.at[t],
                       gsems[t])

    def wait_gather(t, i):
      pltpu.make_async_copy(x_sp.at[src_v.at[t * cps + i]],
                            rows.at[t], gsems[t]).wait()

    def fire_scatter(t, i):
      pltpu.async_copy(rows.at[t], acc.at[dst_v.at[t * cps + i]],
                       ssems[t], add=True)

    def wait_scatter(t, i):
      pltpu.make_async_copy(rows.at[t], acc.at[dst_v.at[t * cps + i]],
                            ssems[t]).wait()

    for p in range(2):
      cslice = pl.ds(p * HALF, HALF)
      @pl.when(s < NS - 1)
      def _():
        pltpu.sync_copy(
            x_hbm.at[pl.ds(s * ROWS_PER_TILE, ROWS_PER_TILE), cslice],
            x_sp.at[rslice])
      @pl.when(s == NS - 1)
      def _():
        pltpu.sync_copy(
            x_hbm.at[pl.ds(15 * ROWS_PER_TILE, X_LAST_TILE), cslice],
            x_sp.at[pl.ds(15 * ROWS_PER_TILE, X_LAST_TILE)])
      pltpu.sync_copy(zer_hbm.at[rslice], acc.at[rslice])
      plsc.subcore_barrier()

      for t in range(NSTREAM):
        fire_gather(t, 0)

      @pl.loop(0, cps)
      def _(i):
        for t in range(NSTREAM):
          wait_gather(t, i)
          fire_scatter(t, i)
        for t in range(NSTREAM):
          wait_scatter(t, i)
          @pl.when(i < cps - 1)
          def _():
            fire_gather(t, i + 1)

      plsc.subcore_barrier()
      pltpu.sync_copy(acc.at[rslice], out_hbm.at[c, p, rslice])
      plsc.subcore_barrier()

  return sc_kernel(x, e3, zeros_init)


def _tc_body(a_hbm, w1_ref, b1_ref, w2_ref, b2_ref, o_ref, a_ref, sem):
  pltpu.async_copy(a_hbm, a_ref, sem).wait()
  a_lo = a_ref[0, 0, :N_NODES] + a_ref[1, 0, :N_NODES]
  a_hi = a_ref[0, 1, :N_NODES] + a_ref[1, 1, :N_NODES]
  w1_lo = w1_ref[:, :HALF]
  w1_hi = w1_ref[:, HALF:]
  h = (lax.dot_general(a_lo, w1_lo, (((1,), (1,)), ((), ())),
                       preferred_element_type=jnp.float32)
       + lax.dot_general(a_hi, w1_hi, (((1,), (1,)), ((), ())),
                         preferred_element_type=jnp.float32))
  h = jnp.maximum(h + b1_ref[...], 0.0)
  o = lax.dot_general(w2_ref[...], h, (((1,), (1,)), ((), ())),
                      preferred_element_type=jnp.float32)
  o_ref[...] = o + b2_ref[...]


def _tc_linear(agg4, W1, b1, W2, b2):
  return pl.pallas_call(
      _tc_body,
      out_shape=jax.ShapeDtypeStruct((DIM, N_NODES), jnp.float32),
      in_specs=[
          pl.BlockSpec(memory_space=pl.ANY),
          pl.BlockSpec((DIM, DIM), lambda: (0, 0)),
          pl.BlockSpec((1, DIM), lambda: (0, 0)),
          pl.BlockSpec((DIM, DIM), lambda: (0, 0)),
          pl.BlockSpec((DIM, 1), lambda: (0, 0)),
      ],
      scratch_shapes=[
          pltpu.VMEM((NC, 2, ACC_ROWS, HALF), jnp.float32),
          pltpu.SemaphoreType.DMA,
      ],
  )(agg4, W1, b1.reshape(1, DIM), W2, b2.reshape(DIM, 1))


def kernel(x, edge_index, W1, b1, W2, b2):
  e3 = edge_index.reshape(2, N_EDGES // CHUNK, CHUNK)
  zeros_init = jnp.zeros((ACC_ROWS, HALF), jnp.float32)
  agg4 = _sc_aggregate(x, e3, zeros_init)
  return _tc_linear(agg4, W1, b1, W2, b2)
